# Initial kernel scaffold; baseline (speedup 1.0000x reference)
import jax
import jax.numpy as jnp
from jax.experimental import pallas as pl


def kernel(rate2_stimulus_set, embedding, w, lower, upper, midpoint, rate):
    raise NotImplementedError("write your pallas kernel here")


# speedup vs baseline: 6.9603x; 6.9603x over previous
import functools

import jax
import jax.numpy as jnp
from jax import lax
from jax.experimental import pallas as pl
from jax.experimental.pallas import tpu as pltpu
from jax.experimental.pallas import tpu_sc as plsc

N_STIMULI = 30
N_DIM = 10
BATCH = 16384

NPAD = 32
DPAD = 16

_NC, _NS = 2, 16
_NW = _NC * _NS
_BPW = BATCH // _NW
_L = 16


def _table_body(e_ref, w_ref, p_ref, t_ref):
    e = e_ref[:, :]
    w = w_ref[:, :]
    ew = e * w
    g = lax.dot_general(ew, e, (((1,), (1,)), ((), ())),
                        preferred_element_type=jnp.float32)
    q = jnp.sum(ew * e, axis=1)
    d2 = q.reshape(32, 1) + q.reshape(1, 32) - 2.0 * g
    d = jnp.sqrt(jnp.maximum(d2, 0.0))
    s = jnp.exp(-3.0 * d)
    lower = p_ref[0]
    upper = p_ref[1]
    midpoint = p_ref[2]
    rate = p_ref[3]
    t_ref[:, :] = lower + (upper - lower) / (1.0 + jnp.exp(-rate * (s - midpoint)))


_table_call = pl.pallas_call(
    _table_body,
    out_shape=jax.ShapeDtypeStruct((NPAD, NPAD), jnp.float32),
    in_specs=[
        pl.BlockSpec(memory_space=pltpu.VMEM),
        pl.BlockSpec(memory_space=pltpu.VMEM),
        pl.BlockSpec(memory_space=pltpu.SMEM),
    ],
)


@functools.lru_cache(maxsize=1)
def _make_gather():
    mesh = plsc.VectorSubcoreMesh(core_axis_name="c", subcore_axis_name="s")

    @functools.partial(
        pl.kernel,
        mesh=mesh,
        out_type=jax.ShapeDtypeStruct((BATCH,), jnp.float32),
        scratch_types=[
            pltpu.VMEM((2 * _BPW,), jnp.int32),
            pltpu.VMEM((NPAD, NPAD), jnp.float32),
            pltpu.VMEM((_BPW,), jnp.float32),
        ],
        compiler_params=pltpu.CompilerParams(needs_layout_passes=False),
    )
    def _gather(table_hbm, idx_hbm, out_hbm, idx_v, tab_v, out_v):
        wid = lax.axis_index("s") * _NC + lax.axis_index("c")
        base = wid * _BPW
        pltpu.sync_copy(idx_hbm.at[pl.ds(2 * base, 2 * _BPW)], idx_v)
        pltpu.sync_copy(table_hbm, tab_v)
        lane = lax.iota(jnp.int32, 16)
        for m in range(_BPW // _L):
            pos = m * (2 * _L) + 2 * lane
            iv = plsc.load_gather(idx_v, [pos])
            jv = plsc.load_gather(idx_v, [pos + 1])
            out_v[pl.ds(m * _L, _L)] = plsc.load_gather(tab_v, [iv, jv])
        pltpu.sync_copy(out_v, out_hbm.at[pl.ds(base, _BPW)])

    return _gather


def kernel(rate2_stimulus_set, embedding, w, lower, upper, midpoint, rate):
    e_pad = jnp.zeros((NPAD, DPAD), jnp.float32).at[:N_STIMULI + 1, :N_DIM].set(embedding)
    w_pad = jnp.zeros((1, DPAD), jnp.float32).at[0, :N_DIM].set(w)
    params = jnp.stack([lower, upper, midpoint, rate]).astype(jnp.float32)
    table = _table_call(e_pad, w_pad, params)
    idx_flat = rate2_stimulus_set.reshape(-1)
    y = _make_gather()(table, idx_flat)
    return y.reshape(BATCH, 1)


# speedup vs baseline: 7.3465x; 1.0555x over previous
import functools

import jax
import jax.numpy as jnp
from jax import lax
from jax.experimental import pallas as pl
from jax.experimental.pallas import tpu as pltpu
from jax.experimental.pallas import tpu_sc as plsc

N_STIMULI = 30
N_DIM = 10
BATCH = 16384
N_PARAMS = N_DIM + 4

NPAD = 32
_NC, _NS, _L = 2, 16, 16
_NW = _NC * _NS
_BPW = BATCH // _NW
_EPS = NPAD * NPAD // _NS
_EMB_PAD = 320


def _rsqrt_newton(x):
    bits = plsc.bitcast(x, jnp.int32)
    y = plsc.bitcast(0x5F3759DF - (bits >> 1), jnp.float32)
    for _ in range(3):
        y = y * (1.5 - 0.5 * x * y * y)
    return y


@functools.lru_cache(maxsize=1)
def _make_sc_kernel():
    mesh = plsc.VectorSubcoreMesh(core_axis_name="c", subcore_axis_name="s")

    @functools.partial(
        pl.kernel,
        mesh=mesh,
        out_type=jax.ShapeDtypeStruct((BATCH,), jnp.float32),
        scratch_types=[
            pltpu.VMEM((2 * _BPW,), jnp.int32),
            pltpu.VMEM((_EMB_PAD,), jnp.float32),
            pltpu.VMEM((N_PARAMS * _L,), jnp.float32),
            pltpu.VMEM((_EPS,), jnp.float32),
            pltpu.VMEM((NPAD * NPAD,), jnp.float32),
            pltpu.VMEM((_BPW,), jnp.float32),
            pltpu.VMEM_SHARED((NPAD * NPAD,), jnp.float32),
            pltpu.SemaphoreType.DMA,
        ],
        compiler_params=pltpu.CompilerParams(needs_layout_passes=False),
    )
    def _sc_kernel(idx_hbm, emb_hbm, pw_hbm, out_hbm,
                   idx_v, emb_v, pw_v, slice_v, tab_v, out_v, shared, sem):
        cid = lax.axis_index("c")
        sid = lax.axis_index("s")
        wid = sid * _NC + cid
        base = wid * _BPW

        idx_cp = pltpu.async_copy(idx_hbm.at[pl.ds(2 * base, 2 * _BPW)],
                                  idx_v, sem)
        pltpu.sync_copy(emb_hbm, emb_v)
        pltpu.sync_copy(pw_hbm, pw_v)

        lane = lax.iota(jnp.int32, _L)

        lower = pw_v[pl.ds(N_DIM * _L, _L)]
        upper = pw_v[pl.ds((N_DIM + 1) * _L, _L)]
        midpt = pw_v[pl.ds((N_DIM + 2) * _L, _L)]
        rate = pw_v[pl.ds((N_DIM + 3) * _L, _L)]
        for v in range(_EPS // _L):
            d_off = sid * 2 + (v >> 1)
            i_raw = lane + (v & 1) * _L
            j_raw = (i_raw + d_off) & (NPAD - 1)
            iv = jnp.minimum(i_raw, N_STIMULI)
            jv = jnp.minimum(j_raw, N_STIMULI)
            d2 = jnp.zeros((_L,), jnp.float32)
            for k in range(N_DIM):
                a = plsc.load_gather(emb_v, [iv * N_DIM + k])
                b = plsc.load_gather(emb_v, [jv * N_DIM + k])
                wk = pw_v[pl.ds(k * _L, _L)]
                diff = a - b
                d2 = d2 + wk * diff * diff
            d = d2 * _rsqrt_newton(d2)
            s = jnp.exp(-3.0 * d)
            slice_v[pl.ds(v * _L, _L)] = (
                lower + (upper - lower) / (1.0 + jnp.exp(-rate * (s - midpt))))
        pltpu.sync_copy(slice_v, shared.at[pl.ds(sid * _EPS, _EPS)])
        plsc.subcore_barrier()
        pltpu.sync_copy(shared, tab_v)

        idx_cp.wait()
        for m in range(_BPW // _L):
            pos = m * (2 * _L) + 2 * lane
            iv = plsc.load_gather(idx_v, [pos])
            jv = plsc.load_gather(idx_v, [pos + 1])
            tpos = ((jv - iv) & (NPAD - 1)) * NPAD + iv
            out_v[pl.ds(m * _L, _L)] = plsc.load_gather(tab_v, [tpos])
        pltpu.sync_copy(out_v, out_hbm.at[pl.ds(base, _BPW)])

    return _sc_kernel


def kernel(rate2_stimulus_set, embedding, w, lower, upper, midpoint, rate):
    params = jnp.concatenate([
        w.astype(jnp.float32),
        jnp.stack([lower, upper, midpoint, rate]).astype(jnp.float32),
    ])
    pw = jnp.broadcast_to(params[:, None], (N_PARAMS, _L)).reshape(-1)
    idx_flat = rate2_stimulus_set.reshape(-1)
    emb_flat = jnp.zeros((_EMB_PAD,), jnp.float32).at[:(N_STIMULI + 1) * N_DIM].set(
        embedding.reshape(-1))
    y = _make_sc_kernel()(idx_flat, emb_flat, pw)
    return y.reshape(BATCH, 1)


# speedup vs baseline: 11.0537x; 1.5046x over previous
import functools

import jax
import jax.numpy as jnp
from jax import lax
from jax.experimental import pallas as pl
from jax.experimental.pallas import tpu as pltpu
from jax.experimental.pallas import tpu_sc as plsc

N_STIMULI = 30
N_DIM = 10
BATCH = 16384
N_PARAMS = N_DIM + 4

NPAD = 32
_NC, _NS, _L = 2, 16, 16
_NW = _NC * _NS
_BPW = BATCH // _NW
_EPS = NPAD * NPAD // _NS
_EMB_PAD = 320


def _rsqrt_newton(x):
    bits = plsc.bitcast(x, jnp.int32)
    y = plsc.bitcast(0x5F3759DF - (bits >> 1), jnp.float32)
    for _ in range(3):
        y = y * (1.5 - 0.5 * x * y * y)
    return y


@functools.lru_cache(maxsize=1)
def _make_sc_kernel():
    mesh = plsc.VectorSubcoreMesh(core_axis_name="c", subcore_axis_name="s")

    @functools.partial(
        pl.kernel,
        mesh=mesh,
        out_type=jax.ShapeDtypeStruct((BATCH,), jnp.float32),
        scratch_types=[
            pltpu.VMEM((2 * _BPW,), jnp.int32),
            pltpu.VMEM((_EMB_PAD,), jnp.float32),
            pltpu.VMEM((N_PARAMS * _L,), jnp.float32),
            pltpu.VMEM((_EPS,), jnp.float32),
            pltpu.VMEM((NPAD * NPAD,), jnp.float32),
            pltpu.VMEM((_BPW,), jnp.float32),
            pltpu.VMEM_SHARED((NPAD * NPAD,), jnp.float32),
            pltpu.SemaphoreType.DMA,
        ],
        compiler_params=pltpu.CompilerParams(needs_layout_passes=False),
    )
    def _sc_kernel(idx_hbm, emb_hbm, pw_hbm, out_hbm,
                   idx_v, emb_v, pw_v, slice_v, tab_v, out_v, shared, sem):
        cid = lax.axis_index("c")
        sid = lax.axis_index("s")
        wid = sid * _NC + cid
        base = wid * _BPW

        idx_cp = pltpu.async_copy(idx_hbm.at[pl.ds(2 * base, 2 * _BPW)],
                                  idx_v, sem)
        pltpu.sync_copy(emb_hbm, emb_v)
        pltpu.sync_copy(pw_hbm, pw_v)

        lane = lax.iota(jnp.int32, _L)

        lower = pw_v[pl.ds(N_DIM * _L, _L)]
        upper = pw_v[pl.ds((N_DIM + 1) * _L, _L)]
        midpt = pw_v[pl.ds((N_DIM + 2) * _L, _L)]
        rate = pw_v[pl.ds((N_DIM + 3) * _L, _L)]
        for v in range(_EPS // _L):
            d_off = sid * 2 + (v >> 1)
            i_raw = lane + (v & 1) * _L
            j_raw = (i_raw + d_off) & (NPAD - 1)
            iv = jnp.minimum(i_raw, N_STIMULI)
            jv = jnp.minimum(j_raw, N_STIMULI)
            d2 = jnp.zeros((_L,), jnp.float32)
            for k in range(N_DIM):
                a = plsc.load_gather(emb_v, [iv * N_DIM + k])
                b = plsc.load_gather(emb_v, [jv * N_DIM + k])
                wk = pw_v[pl.ds(k * _L, _L)]
                diff = a - b
                d2 = d2 + wk * diff * diff
            d = d2 * _rsqrt_newton(d2)
            s = jnp.exp(-3.0 * d)
            slice_v[pl.ds(v * _L, _L)] = (
                lower + (upper - lower) / (1.0 + jnp.exp(-rate * (s - midpt))))
        pltpu.sync_copy(slice_v, shared.at[pl.ds(sid * _EPS, _EPS)])
        plsc.subcore_barrier()
        pltpu.sync_copy(shared, tab_v)

        idx_cp.wait()
        for m in range(_BPW // _L):
            off = (m // 8) * 256 + (m % 8) * _L
            iv = idx_v[pl.ds(off, _L)]
            jv = idx_v[pl.ds(off + 128, _L)]
            tpos = ((jv - iv) & (NPAD - 1)) * NPAD + iv
            out_v[pl.ds(m * _L, _L)] = plsc.load_gather(tab_v, [tpos])
        pltpu.sync_copy(out_v, out_hbm.at[pl.ds(base, _BPW)])

    return _sc_kernel


def kernel(rate2_stimulus_set, embedding, w, lower, upper, midpoint, rate):
    params = jnp.concatenate([
        w.astype(jnp.float32),
        jnp.stack([lower, upper, midpoint, rate]).astype(jnp.float32),
    ])
    pw = jnp.broadcast_to(params[:, None], (N_PARAMS, _L)).reshape(-1)
    idx_flat = rate2_stimulus_set.reshape(128, 128, 2).transpose(0, 2, 1).reshape(-1)
    emb_flat = jnp.zeros((_EMB_PAD,), jnp.float32).at[:(N_STIMULI + 1) * N_DIM].set(
        embedding.reshape(-1))
    y = _make_sc_kernel()(idx_flat, emb_flat, pw)
    return y.reshape(BATCH, 1)


# speedup vs baseline: 11.8438x; 1.0715x over previous
import functools

import jax
import jax.numpy as jnp
from jax import lax
from jax.experimental import pallas as pl
from jax.experimental.pallas import tpu as pltpu
from jax.experimental.pallas import tpu_sc as plsc

N_STIMULI = 30
N_DIM = 10
BATCH = 16384

NPAD = 32
_NC, _NS, _L = 2, 16, 16
_NW = _NC * _NS
_BPW = BATCH // _NW
_EPS = NPAD * NPAD // _NS

_EMB = (N_STIMULI + 1) * N_DIM
_W0 = _EMB
_S0 = _EMB + N_DIM
_PACK = 384
_SM0 = 304
_SMW = 32


def _rsqrt_newton(x):
    bits = plsc.bitcast(x, jnp.int32)
    y = plsc.bitcast(0x5F3759DF - (bits >> 1), jnp.float32)
    for _ in range(3):
        y = y * (1.5 - 0.5 * x * y * y)
    return y


@functools.lru_cache(maxsize=1)
def _make_sc_kernel():
    mesh = plsc.VectorSubcoreMesh(core_axis_name="c", subcore_axis_name="s")

    @functools.partial(
        pl.kernel,
        mesh=mesh,
        out_type=jax.ShapeDtypeStruct((BATCH,), jnp.float32),
        scratch_types=[
            pltpu.VMEM((2 * _BPW,), jnp.int32),
            pltpu.VMEM((_PACK,), jnp.float32),
            pltpu.VMEM((_EPS,), jnp.float32),
            pltpu.VMEM((NPAD * NPAD,), jnp.float32),
            pltpu.VMEM((_BPW,), jnp.float32),
            pltpu.VMEM_SHARED((NPAD * NPAD,), jnp.float32),
            pltpu.SemaphoreType.DMA,
            pltpu.SemaphoreType.DMA,
        ],
        compiler_params=pltpu.CompilerParams(needs_layout_passes=False),
    )
    def _sc_kernel(idx_hbm, pack_hbm, out_hbm,
                   idx_v, data_v, slice_v, tab_v, out_v, shared,
                   sem0, sem1):
        cid = lax.axis_index("c")
        sid = lax.axis_index("s")
        wid = sid * _NC + cid
        base = wid * _BPW

        idx_cp = pltpu.async_copy(idx_hbm.at[pl.ds(2 * base, 2 * _BPW)],
                                  idx_v, sem0)
        pltpu.sync_copy(pack_hbm, data_v)

        lane = lax.iota(jnp.int32, _L)

        def _splat(vec, pos):
            return jnp.sum(jnp.where(lane == pos, vec, 0.0))

        pv1 = data_v[pl.ds(_SM0, _L)]
        pv2 = data_v[pl.ds(_SM0 + _L, _L)]
        wks = [_splat(pv1, _W0 - _SM0 + k) for k in range(N_DIM)]
        lower = _splat(pv2, 0)
        upper = _splat(pv2, 1)
        midpt = _splat(pv2, 2)
        rate = _splat(pv2, 3)
        for v in range(_EPS // _L):
            d_off = sid * 2 + (v >> 1)
            i_raw = lane + (v & 1) * _L
            j_raw = (i_raw + d_off) & (NPAD - 1)
            iv = jnp.minimum(i_raw, N_STIMULI)
            jv = jnp.minimum(j_raw, N_STIMULI)
            d2 = jnp.zeros((_L,), jnp.float32)
            for k in range(N_DIM):
                a = plsc.load_gather(data_v, [iv * N_DIM + k])
                b = plsc.load_gather(data_v, [jv * N_DIM + k])
                wk = wks[k]
                diff = a - b
                d2 = d2 + wk * diff * diff
            d = d2 * _rsqrt_newton(d2)
            s = jnp.exp(-3.0 * d)
            slice_v[pl.ds(v * _L, _L)] = (
                lower + (upper - lower) / (1.0 + jnp.exp(-rate * (s - midpt))))
        pltpu.sync_copy(slice_v, shared.at[pl.ds(sid * _EPS, _EPS)])
        plsc.subcore_barrier()
        pltpu.sync_copy(shared, tab_v)

        idx_cp.wait()
        for m in range(_BPW // _L):
            off = (m // 8) * 256 + (m % 8) * _L
            iv = idx_v[pl.ds(off, _L)]
            jv = idx_v[pl.ds(off + 128, _L)]
            tpos = ((jv - iv) & (NPAD - 1)) * NPAD + iv
            out_v[pl.ds(m * _L, _L)] = plsc.load_gather(tab_v, [tpos])
        pltpu.sync_copy(out_v, out_hbm.at[pl.ds(base, _BPW)])

    return _sc_kernel


def kernel(rate2_stimulus_set, embedding, w, lower, upper, midpoint, rate):
    pack = jnp.concatenate([
        embedding.reshape(-1),
        w.astype(jnp.float32),
        jnp.stack([lower, upper, midpoint, rate]).astype(jnp.float32),
        jnp.zeros((_PACK - _S0 - 4,), jnp.float32),
    ])
    idx_flat = rate2_stimulus_set.reshape(128, 128, 2).transpose(0, 2, 1).reshape(-1)
    y = _make_sc_kernel()(idx_flat, pack)
    return y.reshape(BATCH, 1)
